# baseline (speedup 1.0000x reference)
import functools

import jax
import jax.numpy as jnp
import numpy as np
from jax import lax
from jax.experimental import pallas as pl
from jax.experimental.pallas import tpu as pltpu
from jax.experimental.pallas import tpu_sc as plsc

E = 8
M = 1024
F = 4096
N = 4096
C = 512
TB = 512
NB = N // TB
FD = 512
FB = F // FD

NW = 32
TPW = N // NW
RCH = 32
NCH = TPW // RCH

_TRI = np.tril(np.ones((TB, TB), np.float32), -1)


def _gate_body(tri_ref, x_ref, wg_ref,
               idxd_ref, idxc_ref, scale_ref, cnt_ref, laux_ref,
               cnt_acc, me_acc):
    b = pl.program_id(0)

    @pl.when(b == 0)
    def _init():
        cnt_acc[...] = jnp.zeros_like(cnt_acc)
        me_acc[...] = jnp.zeros_like(me_acc)

    x = x_ref[...]
    logits = lax.dot_general(x, wg_ref[...],
                             (((1,), (1,)), ((), ())),
                             preferred_element_type=jnp.float32)
    mx = jnp.max(logits, axis=1, keepdims=True)
    lane = lax.broadcasted_iota(jnp.int32, (TB, E), 1)
    idx = jnp.min(jnp.where(logits == mx, lane, E), axis=1).astype(jnp.int32)
    eg = jnp.exp(logits - mx)
    gates = eg / jnp.sum(eg, axis=1, keepdims=True)
    mask = (lane == idx[:, None]).astype(jnp.float32)
    gate_s = jnp.sum(gates * mask, axis=1)

    carry = cnt_acc[...]
    locs = lax.dot(tri_ref[...], mask,
                   precision=lax.Precision.HIGHEST) + carry
    loc = jnp.sum(locs * mask, axis=1).astype(jnp.int32)
    cnt_acc[...] = carry + jnp.sum(mask, axis=0, keepdims=True)
    me_acc[...] = me_acc[...] + jnp.sum(gates, axis=0, keepdims=True)

    valid = loc < C
    flat = idx * C + loc
    idxd_ref[...] = jnp.where(valid, flat, N).reshape(1, 1, TB)
    idxc_ref[...] = jnp.where(valid, flat, 0).reshape(1, 1, TB)
    scale = gate_s * valid.astype(jnp.float32)
    scale_ref[...] = jnp.broadcast_to(scale[:, None], (TB, 16))

    @pl.when(b == NB - 1)
    def _fin():
        cnt = cnt_acc[...]
        cnt_ref[...] = cnt
        laux = jnp.sum(me_acc[...] * cnt) * (E / (N * N))
        laux_ref[...] = jnp.broadcast_to(laux, (1, 1))


def _gate(xr, wg):
    tri = jnp.asarray(_TRI)
    return pl.pallas_call(
        _gate_body,
        grid=(NB,),
        in_specs=[
            pl.BlockSpec((TB, TB), lambda b: (0, 0)),
            pl.BlockSpec((TB, M), lambda b: (b, 0)),
            pl.BlockSpec((E, M), lambda b: (0, 0)),
        ],
        out_specs=[
            pl.BlockSpec((1, 1, TB), lambda b: (b, 0, 0)),
            pl.BlockSpec((1, 1, TB), lambda b: (b, 0, 0)),
            pl.BlockSpec((TB, 16), lambda b: (b, 0)),
            pl.BlockSpec((1, E), lambda b: (0, 0)),
            pl.BlockSpec((1, 1), lambda b: (0, 0)),
        ],
        out_shape=[
            jax.ShapeDtypeStruct((NB, 1, TB), jnp.int32),
            jax.ShapeDtypeStruct((NB, 1, TB), jnp.int32),
            jax.ShapeDtypeStruct((N, 16), jnp.float32),
            jax.ShapeDtypeStruct((1, E), jnp.float32),
            jax.ShapeDtypeStruct((1, 1), jnp.float32),
        ],
        scratch_shapes=[
            pltpu.VMEM((1, E), jnp.float32),
            pltpu.VMEM((1, E), jnp.float32),
        ],
        compiler_params=pltpu.CompilerParams(
            dimension_semantics=("arbitrary",)),
    )(tri, xr, wg)


_SC_MESH = plsc.VectorSubcoreMesh(core_axis_name="c", subcore_axis_name="s")


@functools.partial(
    pl.kernel,
    mesh=_SC_MESH,
    out_type=jax.ShapeDtypeStruct((N + 8, M), jnp.float32),
    scratch_types=[
        pltpu.VMEM((NCH, RCH), jnp.int32),
        pltpu.VMEM((RCH, M), jnp.float32),
        pltpu.SemaphoreType.DMA,
    ],
)
def _dispatch(x_hbm, idx_hbm, disp_hbm, idx_v, buf_v, sem):
    wid = lax.axis_index("s") * 2 + lax.axis_index("c")
    base = wid * TPW
    pltpu.sync_copy(idx_hbm.at[wid], idx_v)
    for ch in range(NCH):
        pltpu.sync_copy(x_hbm.at[pl.ds(base + ch * RCH, RCH)], buf_v)
        pltpu.async_copy(buf_v, disp_hbm.at[idx_v.at[ch]], sem).wait()


def _ffn_body(cnt_ref, x_ref, w1_ref, w2_ref, out_ref, acc):
    e = pl.program_id(0)
    f = pl.program_id(1)
    cnt = cnt_ref[0, e].astype(jnp.int32)
    row = lax.broadcasted_iota(jnp.int32, (C, 1), 0)
    x = x_ref[0] * (row < cnt).astype(jnp.float32)
    h = jnp.maximum(
        lax.dot(x, w1_ref[0], preferred_element_type=jnp.float32), 0.0)
    p = lax.dot(h, w2_ref[0], preferred_element_type=jnp.float32)

    @pl.when(f == 0)
    def _first():
        acc[...] = p

    @pl.when(f != 0)
    def _rest():
        acc[...] = acc[...] + p

    @pl.when(f == FB - 1)
    def _fin():
        out_ref[0] = acc[...]


def _ffn(cnt, disp3, W1, W2):
    return pl.pallas_call(
        _ffn_body,
        grid=(E, FB),
        in_specs=[
            pl.BlockSpec(memory_space=pltpu.SMEM),
            pl.BlockSpec((1, C, M), lambda e, f: (e, 0, 0)),
            pl.BlockSpec((1, M, FD), lambda e, f: (e, 0, f)),
            pl.BlockSpec((1, FD, M), lambda e, f: (e, f, 0)),
        ],
        out_specs=pl.BlockSpec((1, C, M), lambda e, f: (e, 0, 0)),
        out_shape=jax.ShapeDtypeStruct((E, C, M), jnp.float32),
        scratch_shapes=[pltpu.VMEM((C, M), jnp.float32)],
        compiler_params=pltpu.CompilerParams(
            dimension_semantics=("arbitrary", "arbitrary")),
    )(cnt, disp3, W1, W2)


@functools.partial(
    pl.kernel,
    mesh=_SC_MESH,
    out_type=jax.ShapeDtypeStruct((N, M), jnp.float32),
    scratch_types=[
        pltpu.VMEM((NCH, RCH), jnp.int32),
        pltpu.VMEM((RCH, M), jnp.float32),
        pltpu.SemaphoreType.DMA,
    ],
)
def _combine(eo_hbm, idx_hbm, out_hbm, idx_v, buf_v, sem):
    wid = lax.axis_index("s") * 2 + lax.axis_index("c")
    base = wid * TPW
    pltpu.sync_copy(idx_hbm.at[wid], idx_v)
    for ch in range(NCH):
        pltpu.async_copy(eo_hbm.at[idx_v.at[ch]], buf_v, sem).wait()
        pltpu.sync_copy(buf_v, out_hbm.at[pl.ds(base + ch * RCH, RCH)])


def _scale_body(g_ref, s_ref, out_ref):
    out_ref[...] = g_ref[...] * s_ref[...][:, :1]


def _scale(gathered, scale_b):
    return pl.pallas_call(
        _scale_body,
        grid=(NB,),
        in_specs=[
            pl.BlockSpec((TB, M), lambda b: (b, 0)),
            pl.BlockSpec((TB, 16), lambda b: (b, 0)),
        ],
        out_specs=pl.BlockSpec((TB, M), lambda b: (b, 0)),
        out_shape=jax.ShapeDtypeStruct((N, M), jnp.float32),
    )(gathered, scale_b)


def kernel(x, wg, W1, W2):
    S0, T0, _ = x.shape
    xr = x.reshape(N, M)
    idxd3, idxc3, scale_b, cnt, laux = _gate(xr, wg)
    idxd = idxd3.reshape(NW, NCH, RCH)
    idxc = idxc3.reshape(NW, NCH, RCH)
    disp = _dispatch(xr, idxd)
    disp3 = disp[:N].reshape(E, C, M)
    eo = _ffn(cnt, disp3, W1, W2).reshape(N, M)
    gathered = _combine(eo, idxc)
    combined = _scale(gathered, scale_b).reshape(S0, T0, M)
    return combined, laux.reshape(())


# speedup vs baseline: 1.0213x; 1.0213x over previous
import functools

import jax
import jax.numpy as jnp
import numpy as np
from jax import lax
from jax.experimental import pallas as pl
from jax.experimental.pallas import tpu as pltpu
from jax.experimental.pallas import tpu_sc as plsc

E = 8
M = 1024
F = 4096
N = 4096
C = 512
TB = 512
NB = N // TB
FD = 512
FB = F // FD

NW = 32
TPW = N // NW
RCH = 32
NCH = TPW // RCH

_TRI = np.tril(np.ones((TB, TB), np.float32), -1)


def _gate_body(tri_ref, x_ref, wg_ref,
               idxd_ref, idxc_ref, scale_ref, cnt_ref, laux_ref,
               cnt_acc, me_acc):
    b = pl.program_id(0)

    @pl.when(b == 0)
    def _init():
        cnt_acc[...] = jnp.zeros_like(cnt_acc)
        me_acc[...] = jnp.zeros_like(me_acc)

    x = x_ref[...]
    logits = lax.dot_general(x, wg_ref[...],
                             (((1,), (1,)), ((), ())),
                             preferred_element_type=jnp.float32)
    mx = jnp.max(logits, axis=1, keepdims=True)
    lane = lax.broadcasted_iota(jnp.int32, (TB, E), 1)
    idx = jnp.min(jnp.where(logits == mx, lane, E), axis=1).astype(jnp.int32)
    eg = jnp.exp(logits - mx)
    gates = eg / jnp.sum(eg, axis=1, keepdims=True)
    mask = (lane == idx[:, None]).astype(jnp.float32)
    gate_s = jnp.sum(gates * mask, axis=1)

    carry = cnt_acc[...]
    locs = lax.dot(tri_ref[...], mask,
                   precision=lax.Precision.HIGHEST) + carry
    loc = jnp.sum(locs * mask, axis=1).astype(jnp.int32)
    cnt_acc[...] = carry + jnp.sum(mask, axis=0, keepdims=True)
    me_acc[...] = me_acc[...] + jnp.sum(gates, axis=0, keepdims=True)

    valid = loc < C
    flat = idx * C + loc
    idxd_ref[...] = jnp.where(valid, flat, N).reshape(1, 1, TB)
    idxc_ref[...] = jnp.where(valid, flat, 0).reshape(1, 1, TB)
    scale = gate_s * valid.astype(jnp.float32)
    scale_ref[...] = jnp.broadcast_to(scale[:, None], (TB, 16))

    @pl.when(b == NB - 1)
    def _fin():
        cnt = cnt_acc[...]
        cnt_ref[...] = cnt
        laux = jnp.sum(me_acc[...] * cnt) * (E / (N * N))
        laux_ref[...] = jnp.broadcast_to(laux, (1, 1))


def _gate(xr, wg):
    tri = jnp.asarray(_TRI)
    return pl.pallas_call(
        _gate_body,
        grid=(NB,),
        in_specs=[
            pl.BlockSpec((TB, TB), lambda b: (0, 0)),
            pl.BlockSpec((TB, M), lambda b: (b, 0)),
            pl.BlockSpec((E, M), lambda b: (0, 0)),
        ],
        out_specs=[
            pl.BlockSpec((1, 1, TB), lambda b: (b, 0, 0)),
            pl.BlockSpec((1, 1, TB), lambda b: (b, 0, 0)),
            pl.BlockSpec((TB, 16), lambda b: (b, 0)),
            pl.BlockSpec((1, E), lambda b: (0, 0)),
            pl.BlockSpec((1, 1), lambda b: (0, 0)),
        ],
        out_shape=[
            jax.ShapeDtypeStruct((NB, 1, TB), jnp.int32),
            jax.ShapeDtypeStruct((NB, 1, TB), jnp.int32),
            jax.ShapeDtypeStruct((N, 16), jnp.float32),
            jax.ShapeDtypeStruct((1, E), jnp.float32),
            jax.ShapeDtypeStruct((1, 1), jnp.float32),
        ],
        scratch_shapes=[
            pltpu.VMEM((1, E), jnp.float32),
            pltpu.VMEM((1, E), jnp.float32),
        ],
        compiler_params=pltpu.CompilerParams(
            dimension_semantics=("arbitrary",)),
    )(tri, xr, wg)


_SC_MESH = plsc.VectorSubcoreMesh(core_axis_name="c", subcore_axis_name="s")


@functools.partial(
    pl.kernel,
    mesh=_SC_MESH,
    out_type=jax.ShapeDtypeStruct((N + 8, M), jnp.float32),
    scratch_types=[
        pltpu.VMEM((NCH, RCH), jnp.int32),
        pltpu.VMEM((RCH, M), jnp.float32),
        pltpu.SemaphoreType.DMA,
    ],
)
def _dispatch(x_hbm, idx_hbm, disp_hbm, idx_v, buf_v, sem):
    wid = lax.axis_index("s") * 2 + lax.axis_index("c")
    base = wid * TPW
    pltpu.sync_copy(idx_hbm.at[wid], idx_v)
    for ch in range(NCH):
        pltpu.sync_copy(x_hbm.at[pl.ds(base + ch * RCH, RCH)], buf_v)
        pltpu.async_copy(buf_v, disp_hbm.at[idx_v.at[ch]], sem).wait()


def _ffn_body(cnt_ref, x_ref, w1_ref, w2_ref, out_ref, acc):
    e = pl.program_id(0)
    f = pl.program_id(1)
    cnt = cnt_ref[0, e].astype(jnp.int32)
    row = lax.broadcasted_iota(jnp.int32, (C, 1), 0)
    x = x_ref[0] * (row < cnt).astype(jnp.float32)
    h = jnp.maximum(
        lax.dot(x.astype(jnp.bfloat16), w1_ref[0].astype(jnp.bfloat16),
                preferred_element_type=jnp.float32), 0.0)
    p = lax.dot(h.astype(jnp.bfloat16), w2_ref[0].astype(jnp.bfloat16),
                preferred_element_type=jnp.float32)

    @pl.when(f == 0)
    def _first():
        acc[...] = p

    @pl.when(f != 0)
    def _rest():
        acc[...] = acc[...] + p

    @pl.when(f == FB - 1)
    def _fin():
        out_ref[0] = acc[...]


def _ffn(cnt, disp3, W1, W2):
    return pl.pallas_call(
        _ffn_body,
        grid=(E, FB),
        in_specs=[
            pl.BlockSpec(memory_space=pltpu.SMEM),
            pl.BlockSpec((1, C, M), lambda e, f: (e, 0, 0)),
            pl.BlockSpec((1, M, FD), lambda e, f: (e, 0, f)),
            pl.BlockSpec((1, FD, M), lambda e, f: (e, f, 0)),
        ],
        out_specs=pl.BlockSpec((1, C, M), lambda e, f: (e, 0, 0)),
        out_shape=jax.ShapeDtypeStruct((E, C, M), jnp.float32),
        scratch_shapes=[pltpu.VMEM((C, M), jnp.float32)],
        compiler_params=pltpu.CompilerParams(
            dimension_semantics=("arbitrary", "arbitrary")),
    )(cnt, disp3, W1, W2)


@functools.partial(
    pl.kernel,
    mesh=_SC_MESH,
    out_type=jax.ShapeDtypeStruct((N, M), jnp.float32),
    scratch_types=[
        pltpu.VMEM((NCH, RCH), jnp.int32),
        pltpu.VMEM((TPW, 16), jnp.float32),
        pltpu.VMEM((RCH, M), jnp.float32),
        pltpu.SemaphoreType.DMA,
    ],
)
def _combine(eo_hbm, idx_hbm, scl_hbm, out_hbm, idx_v, scl_v, buf_v, sem):
    wid = lax.axis_index("s") * 2 + lax.axis_index("c")
    base = wid * TPW
    pltpu.sync_copy(idx_hbm.at[wid], idx_v)
    pltpu.sync_copy(scl_hbm.at[wid], scl_v)
    for ch in range(NCH):
        pltpu.async_copy(eo_hbm.at[idx_v.at[ch]], buf_v, sem).wait()

        def _row(r, _, ch=ch):
            sv = scl_v[ch * RCH + r]
            for j in range(M // 16):
                buf_v[r, pl.ds(j * 16, 16)] = buf_v[r, pl.ds(j * 16, 16)] * sv
            return 0

        lax.fori_loop(0, RCH, _row, 0)
        pltpu.sync_copy(buf_v, out_hbm.at[pl.ds(base + ch * RCH, RCH)])


def kernel(x, wg, W1, W2):
    S0, T0, _ = x.shape
    xr = x.reshape(N, M)
    idxd3, idxc3, scale_b, cnt, laux = _gate(xr, wg)
    idxd = idxd3.reshape(NW, NCH, RCH)
    idxc = idxc3.reshape(NW, NCH, RCH)
    disp = _dispatch(xr, idxd)
    disp3 = disp[:N].reshape(E, C, M)
    eo = _ffn(cnt, disp3, W1, W2).reshape(N, M)
    combined = _combine(eo, idxc, scale_b.reshape(NW, TPW, 16)).reshape(S0, T0, M)
    return combined, laux.reshape(())


# speedup vs baseline: 1.2035x; 1.1784x over previous
import functools

import jax
import jax.numpy as jnp
import numpy as np
from jax import lax
from jax.experimental import pallas as pl
from jax.experimental.pallas import tpu as pltpu
from jax.experimental.pallas import tpu_sc as plsc

E = 8
M = 1024
F = 4096
N = 4096
C = 512
TB = 512
NB = N // TB
FD = 1024
FB = F // FD

NW = 32
TPW = N // NW
RCH = 32
NCH = TPW // RCH

_TRI = np.tril(np.ones((TB, TB), np.float32), -1)


def _gate_body(tri_ref, x_ref, wg_ref,
               idxd_ref, idxc_ref, scale_ref, cnt_ref, laux_ref,
               cnt_acc, me_acc):
    b = pl.program_id(0)

    @pl.when(b == 0)
    def _init():
        cnt_acc[...] = jnp.zeros_like(cnt_acc)
        me_acc[...] = jnp.zeros_like(me_acc)

    x = x_ref[...]
    logits = lax.dot_general(x.astype(jnp.bfloat16),
                             wg_ref[...].astype(jnp.bfloat16),
                             (((1,), (1,)), ((), ())),
                             preferred_element_type=jnp.float32)
    mx = jnp.max(logits, axis=1, keepdims=True)
    lane = lax.broadcasted_iota(jnp.int32, (TB, E), 1)
    idx = jnp.min(jnp.where(logits == mx, lane, E), axis=1).astype(jnp.int32)
    eg = jnp.exp(logits - mx)
    gates = eg / jnp.sum(eg, axis=1, keepdims=True)
    mask = (lane == idx[:, None]).astype(jnp.float32)
    gate_s = jnp.sum(gates * mask, axis=1)

    carry = cnt_acc[...]
    locs = lax.dot(tri_ref[...], mask.astype(jnp.bfloat16),
                   preferred_element_type=jnp.float32) + carry
    loc = jnp.sum(locs * mask, axis=1).astype(jnp.int32)
    cnt_acc[...] = carry + jnp.sum(mask, axis=0, keepdims=True)
    me_acc[...] = me_acc[...] + jnp.sum(gates, axis=0, keepdims=True)

    valid = loc < C
    idxd_ref[...] = (idx * (C + 1) + jnp.minimum(loc, C)).reshape(1, 1, TB)
    idxc_ref[...] = jnp.where(valid, idx * C + loc, 0).reshape(1, 1, TB)
    scale = gate_s * valid.astype(jnp.float32)
    scale_ref[...] = jnp.broadcast_to(scale[:, None], (TB, 16))

    @pl.when(b == NB - 1)
    def _fin():
        cnt = cnt_acc[...]
        cnt_ref[...] = cnt
        laux = jnp.sum(me_acc[...] * cnt) * (E / (N * N))
        laux_ref[...] = jnp.broadcast_to(laux, (1, 1))


def _gate(xr, wg):
    tri = jnp.asarray(_TRI, dtype=jnp.bfloat16)
    return pl.pallas_call(
        _gate_body,
        grid=(NB,),
        in_specs=[
            pl.BlockSpec((TB, TB), lambda b: (0, 0)),
            pl.BlockSpec((TB, M), lambda b: (b, 0)),
            pl.BlockSpec((E, M), lambda b: (0, 0)),
        ],
        out_specs=[
            pl.BlockSpec((1, 1, TB), lambda b: (b, 0, 0)),
            pl.BlockSpec((1, 1, TB), lambda b: (b, 0, 0)),
            pl.BlockSpec((TB, 16), lambda b: (b, 0)),
            pl.BlockSpec((1, E), lambda b: (0, 0)),
            pl.BlockSpec((1, 1), lambda b: (0, 0)),
        ],
        out_shape=[
            jax.ShapeDtypeStruct((NB, 1, TB), jnp.int32),
            jax.ShapeDtypeStruct((NB, 1, TB), jnp.int32),
            jax.ShapeDtypeStruct((N, 16), jnp.float32),
            jax.ShapeDtypeStruct((1, E), jnp.float32),
            jax.ShapeDtypeStruct((1, 1), jnp.float32),
        ],
        scratch_shapes=[
            pltpu.VMEM((1, E), jnp.float32),
            pltpu.VMEM((1, E), jnp.float32),
        ],
        compiler_params=pltpu.CompilerParams(
            dimension_semantics=("arbitrary",)),
    )(tri, xr, wg)


_SC_MESH = plsc.VectorSubcoreMesh(core_axis_name="c", subcore_axis_name="s")


@functools.partial(
    pl.kernel,
    mesh=_SC_MESH,
    out_type=jax.ShapeDtypeStruct((E * (C + 1), M), jnp.float32),
    scratch_types=[
        pltpu.VMEM((NCH, RCH), jnp.int32),
        pltpu.VMEM((RCH, M), jnp.float32),
        pltpu.VMEM((RCH, M), jnp.float32),
        pltpu.SemaphoreType.DMA,
        pltpu.SemaphoreType.DMA,
        pltpu.SemaphoreType.DMA,
        pltpu.SemaphoreType.DMA,
    ],
)
def _dispatch(x_hbm, idx_hbm, disp_hbm, idx_v, buf0, buf1, l0, l1, s0, s1):
    bufs = [buf0, buf1]
    lsems = [l0, l1]
    ssems = [s0, s1]
    wid = lax.axis_index("s") * 2 + lax.axis_index("c")
    base = wid * TPW
    pltpu.sync_copy(idx_hbm.at[wid], idx_v)
    loads = {0: pltpu.async_copy(x_hbm.at[pl.ds(base, RCH)], buf0, l0)}
    scats = {}
    for ch in range(NCH):
        b = ch % 2
        nb = (ch + 1) % 2
        if ch + 1 < NCH:
            if ch - 1 >= 0:
                scats[ch - 1].wait()
            loads[ch + 1] = pltpu.async_copy(
                x_hbm.at[pl.ds(base + (ch + 1) * RCH, RCH)], bufs[nb],
                lsems[nb])
        loads[ch].wait()
        scats[ch] = pltpu.async_copy(bufs[b], disp_hbm.at[idx_v.at[ch]],
                                     ssems[b])
    scats[NCH - 2].wait()
    scats[NCH - 1].wait()


def _ffn_body(cnt_ref, x_ref, w1_ref, w2_ref, out_ref, acc):
    e = pl.program_id(0)
    f = pl.program_id(1)
    cnt = cnt_ref[0, e].astype(jnp.int32)
    row = lax.broadcasted_iota(jnp.int32, (C, 1), 0)
    x = x_ref[0] * (row < cnt).astype(jnp.float32)
    h = jnp.maximum(
        lax.dot(x.astype(jnp.bfloat16), w1_ref[0].astype(jnp.bfloat16),
                preferred_element_type=jnp.float32), 0.0)
    p = lax.dot(h.astype(jnp.bfloat16), w2_ref[0].astype(jnp.bfloat16),
                preferred_element_type=jnp.float32)

    @pl.when(f == 0)
    def _first():
        acc[...] = p

    @pl.when(f != 0)
    def _rest():
        acc[...] = acc[...] + p

    @pl.when(f == FB - 1)
    def _fin():
        out_ref[0] = acc[...]


def _ffn(cnt, disp3, W1, W2):
    return pl.pallas_call(
        _ffn_body,
        grid=(E, FB),
        in_specs=[
            pl.BlockSpec(memory_space=pltpu.SMEM),
            pl.BlockSpec((1, C, M), lambda e, f: (e, 0, 0)),
            pl.BlockSpec((1, M, FD), lambda e, f: (e, 0, f)),
            pl.BlockSpec((1, FD, M), lambda e, f: (e, f, 0)),
        ],
        out_specs=pl.BlockSpec((1, C, M), lambda e, f: (e, 0, 0)),
        out_shape=jax.ShapeDtypeStruct((E, C, M), jnp.float32),
        scratch_shapes=[pltpu.VMEM((C, M), jnp.float32)],
        compiler_params=pltpu.CompilerParams(
            dimension_semantics=("arbitrary", "arbitrary")),
    )(cnt, disp3, W1, W2)


@functools.partial(
    pl.kernel,
    mesh=_SC_MESH,
    out_type=jax.ShapeDtypeStruct((N, M), jnp.float32),
    scratch_types=[
        pltpu.VMEM((NCH, RCH), jnp.int32),
        pltpu.VMEM((TPW, 16), jnp.float32),
        pltpu.VMEM((RCH, M), jnp.float32),
        pltpu.VMEM((RCH, M), jnp.float32),
        pltpu.SemaphoreType.DMA,
        pltpu.SemaphoreType.DMA,
        pltpu.SemaphoreType.DMA,
        pltpu.SemaphoreType.DMA,
    ],
)
def _combine(eo_hbm, idx_hbm, scl_hbm, out_hbm,
             idx_v, scl_v, buf0, buf1, g0, g1, s0, s1):
    bufs = [buf0, buf1]
    gsems = [g0, g1]
    ssems = [s0, s1]
    wid = lax.axis_index("s") * 2 + lax.axis_index("c")
    base = wid * TPW
    pltpu.sync_copy(idx_hbm.at[wid], idx_v)
    pltpu.sync_copy(scl_hbm.at[wid], scl_v)
    gathers = {0: pltpu.async_copy(eo_hbm.at[idx_v.at[0]], buf0, g0)}
    stores = {}
    for ch in range(NCH):
        b = ch % 2
        nb = (ch + 1) % 2
        if ch + 1 < NCH:
            if ch - 1 >= 0:
                stores[ch - 1].wait()
            gathers[ch + 1] = pltpu.async_copy(
                eo_hbm.at[idx_v.at[ch + 1]], bufs[nb], gsems[nb])
        gathers[ch].wait()
        buf = bufs[b]

        def _row(r, _, ch=ch, buf=buf):
            sv = scl_v[ch * RCH + r]
            for j in range(M // 16):
                buf[r, pl.ds(j * 16, 16)] = buf[r, pl.ds(j * 16, 16)] * sv
            return 0

        lax.fori_loop(0, RCH, _row, 0)
        stores[ch] = pltpu.async_copy(
            buf, out_hbm.at[pl.ds(base + ch * RCH, RCH)], ssems[b])
    stores[NCH - 2].wait()
    stores[NCH - 1].wait()


def kernel(x, wg, W1, W2):
    S0, T0, _ = x.shape
    xr = x.reshape(N, M)
    idxd3, idxc3, scale_b, cnt, laux = _gate(xr, wg)
    idxd = idxd3.reshape(NW, NCH, RCH)
    idxc = idxc3.reshape(NW, NCH, RCH)
    disp3 = _dispatch(xr, idxd).reshape(E, C + 1, M)
    eo = _ffn(cnt, disp3, W1, W2).reshape(N, M)
    combined = _combine(eo, idxc, scale_b.reshape(NW, TPW, 16)).reshape(S0, T0, M)
    return combined, laux.reshape(())


# speedup vs baseline: 1.2714x; 1.0564x over previous
import functools

import jax
import jax.numpy as jnp
import numpy as np
from jax import lax
from jax.experimental import pallas as pl
from jax.experimental.pallas import tpu as pltpu
from jax.experimental.pallas import tpu_sc as plsc

E = 8
M = 1024
F = 4096
N = 4096
C = 512
TB = 1024
NB = N // TB
FD = 1024
FB = F // FD

NW = 32
TPW = N // NW
RCH = 32
NCH = TPW // RCH

_TRI = np.tril(np.ones((TB, TB), np.float32), -1)


def _gate_body(tri_ref, x_ref, wg_ref,
               idxd_ref, idxc_ref, scale_ref, cnt_ref, laux_ref,
               cnt_acc, me_acc):
    b = pl.program_id(0)

    @pl.when(b == 0)
    def _init():
        cnt_acc[...] = jnp.zeros_like(cnt_acc)
        me_acc[...] = jnp.zeros_like(me_acc)

    x = x_ref[...]
    logits = lax.dot_general(x.astype(jnp.bfloat16),
                             wg_ref[...].astype(jnp.bfloat16),
                             (((1,), (1,)), ((), ())),
                             preferred_element_type=jnp.float32)
    mx = jnp.max(logits, axis=1, keepdims=True)
    lane = lax.broadcasted_iota(jnp.int32, (TB, E), 1)
    idx = jnp.min(jnp.where(logits == mx, lane, E), axis=1).astype(jnp.int32)
    eg = jnp.exp(logits - mx)
    gates = eg / jnp.sum(eg, axis=1, keepdims=True)
    mask = (lane == idx[:, None]).astype(jnp.float32)
    gate_s = jnp.sum(gates * mask, axis=1)

    carry = cnt_acc[...]
    locs = lax.dot(tri_ref[...], mask.astype(jnp.bfloat16),
                   preferred_element_type=jnp.float32) + carry
    loc = jnp.sum(locs * mask, axis=1).astype(jnp.int32)
    cnt_acc[...] = carry + jnp.sum(mask, axis=0, keepdims=True)
    me_acc[...] = me_acc[...] + jnp.sum(gates, axis=0, keepdims=True)

    valid = loc < C
    flat = idx * C + loc
    idxd_ref[...] = jnp.where(valid, flat, N).reshape(1, 1, TB)
    idxc_ref[...] = jnp.where(valid, idx * C + loc, 0).reshape(1, 1, TB)
    scale = gate_s * valid.astype(jnp.float32)
    scale_ref[...] = jnp.broadcast_to(scale[:, None], (TB, 16))

    @pl.when(b == NB - 1)
    def _fin():
        cnt = cnt_acc[...]
        cnt_ref[...] = cnt
        laux = jnp.sum(me_acc[...] * cnt) * (E / (N * N))
        laux_ref[...] = jnp.broadcast_to(laux, (1, 1))


def _gate(xr, wg):
    tri = jnp.asarray(_TRI, dtype=jnp.bfloat16)
    return pl.pallas_call(
        _gate_body,
        grid=(NB,),
        in_specs=[
            pl.BlockSpec((TB, TB), lambda b: (0, 0)),
            pl.BlockSpec((TB, M), lambda b: (b, 0)),
            pl.BlockSpec((E, M), lambda b: (0, 0)),
        ],
        out_specs=[
            pl.BlockSpec((1, 1, TB), lambda b: (b, 0, 0)),
            pl.BlockSpec((1, 1, TB), lambda b: (b, 0, 0)),
            pl.BlockSpec((TB, 16), lambda b: (b, 0)),
            pl.BlockSpec((1, E), lambda b: (0, 0)),
            pl.BlockSpec((1, 1), lambda b: (0, 0)),
        ],
        out_shape=[
            jax.ShapeDtypeStruct((NB, 1, TB), jnp.int32),
            jax.ShapeDtypeStruct((NB, 1, TB), jnp.int32),
            jax.ShapeDtypeStruct((N, 16), jnp.float32),
            jax.ShapeDtypeStruct((1, E), jnp.float32),
            jax.ShapeDtypeStruct((1, 1), jnp.float32),
        ],
        scratch_shapes=[
            pltpu.VMEM((1, E), jnp.float32),
            pltpu.VMEM((1, E), jnp.float32),
        ],
        compiler_params=pltpu.CompilerParams(
            dimension_semantics=("arbitrary",)),
    )(tri, xr, wg)


_SC_MESH = plsc.VectorSubcoreMesh(core_axis_name="c", subcore_axis_name="s")


@functools.partial(
    pl.kernel,
    mesh=_SC_MESH,
    out_type=jax.ShapeDtypeStruct((N + 8, M), jnp.float32),
    scratch_types=[
        pltpu.VMEM((NCH, RCH), jnp.int32),
        pltpu.VMEM((RCH, M), jnp.float32),
        pltpu.VMEM((RCH, M), jnp.float32),
        pltpu.SemaphoreType.DMA,
        pltpu.SemaphoreType.DMA,
        pltpu.SemaphoreType.DMA,
        pltpu.SemaphoreType.DMA,
    ],
)
def _dispatch(x_hbm, idx_hbm, disp_hbm, idx_v, buf0, buf1, l0, l1, s0, s1):
    bufs = [buf0, buf1]
    lsems = [l0, l1]
    ssems = [s0, s1]
    wid = lax.axis_index("s") * 2 + lax.axis_index("c")
    base = wid * TPW
    pltpu.sync_copy(idx_hbm.at[wid], idx_v)
    loads = {0: pltpu.async_copy(x_hbm.at[pl.ds(base, RCH)], buf0, l0)}
    scats = {}
    for ch in range(NCH):
        b = ch % 2
        nb = (ch + 1) % 2
        if ch + 1 < NCH:
            if ch - 1 >= 0:
                scats[ch - 1].wait()
            loads[ch + 1] = pltpu.async_copy(
                x_hbm.at[pl.ds(base + (ch + 1) * RCH, RCH)], bufs[nb],
                lsems[nb])
        loads[ch].wait()
        scats[ch] = pltpu.async_copy(bufs[b], disp_hbm.at[idx_v.at[ch]],
                                     ssems[b])
    scats[NCH - 2].wait()
    scats[NCH - 1].wait()


def _ffn_body(cnt_ref, x_ref, w1_ref, w2_ref, out_ref, acc):
    e = pl.program_id(0)
    f = pl.program_id(1)
    cnt = cnt_ref[0, e].astype(jnp.int32)
    row = lax.broadcasted_iota(jnp.int32, (C, 1), 0)
    x = x_ref[...] * (row < cnt).astype(jnp.float32)
    h = jnp.maximum(
        lax.dot(x.astype(jnp.bfloat16), w1_ref[0].astype(jnp.bfloat16),
                preferred_element_type=jnp.float32), 0.0)
    p = lax.dot(h.astype(jnp.bfloat16), w2_ref[0].astype(jnp.bfloat16),
                preferred_element_type=jnp.float32)

    @pl.when(f == 0)
    def _first():
        acc[...] = p

    @pl.when(f != 0)
    def _rest():
        acc[...] = acc[...] + p

    @pl.when(f == FB - 1)
    def _fin():
        out_ref[...] = acc[...]


def _ffn(cnt, disp, W1, W2):
    return pl.pallas_call(
        _ffn_body,
        grid=(E, FB),
        in_specs=[
            pl.BlockSpec(memory_space=pltpu.SMEM),
            pl.BlockSpec((C, M), lambda e, f: (e, 0)),
            pl.BlockSpec((1, M, FD), lambda e, f: (e, 0, f)),
            pl.BlockSpec((1, FD, M), lambda e, f: (e, f, 0)),
        ],
        out_specs=pl.BlockSpec((C, M), lambda e, f: (e, 0)),
        out_shape=jax.ShapeDtypeStruct((N, M), jnp.float32),
        scratch_shapes=[pltpu.VMEM((C, M), jnp.float32)],
        compiler_params=pltpu.CompilerParams(
            dimension_semantics=("arbitrary", "arbitrary")),
    )(cnt, disp, W1, W2)


@functools.partial(
    pl.kernel,
    mesh=_SC_MESH,
    out_type=jax.ShapeDtypeStruct((N, M), jnp.float32),
    scratch_types=[
        pltpu.VMEM((NCH, RCH), jnp.int32),
        pltpu.VMEM((TPW, 16), jnp.float32),
        pltpu.VMEM((RCH, M), jnp.float32),
        pltpu.VMEM((RCH, M), jnp.float32),
        pltpu.SemaphoreType.DMA,
        pltpu.SemaphoreType.DMA,
        pltpu.SemaphoreType.DMA,
        pltpu.SemaphoreType.DMA,
    ],
)
def _combine(eo_hbm, idx_hbm, scl_hbm, out_hbm,
             idx_v, scl_v, buf0, buf1, g0, g1, s0, s1):
    bufs = [buf0, buf1]
    gsems = [g0, g1]
    ssems = [s0, s1]
    wid = lax.axis_index("s") * 2 + lax.axis_index("c")
    base = wid * TPW
    pltpu.sync_copy(idx_hbm.at[wid], idx_v)
    pltpu.sync_copy(scl_hbm.at[wid], scl_v)
    gathers = {0: pltpu.async_copy(eo_hbm.at[idx_v.at[0]], buf0, g0)}
    stores = {}
    for ch in range(NCH):
        b = ch % 2
        nb = (ch + 1) % 2
        if ch + 1 < NCH:
            if ch - 1 >= 0:
                stores[ch - 1].wait()
            gathers[ch + 1] = pltpu.async_copy(
                eo_hbm.at[idx_v.at[ch + 1]], bufs[nb], gsems[nb])
        gathers[ch].wait()
        buf = bufs[b]

        def _row(r, _, ch=ch, buf=buf):
            sv = scl_v[ch * RCH + r]
            for j in range(M // 16):
                buf[r, pl.ds(j * 16, 16)] = buf[r, pl.ds(j * 16, 16)] * sv
            return 0

        lax.fori_loop(0, RCH, _row, 0)
        stores[ch] = pltpu.async_copy(
            buf, out_hbm.at[pl.ds(base + ch * RCH, RCH)], ssems[b])
    stores[NCH - 2].wait()
    stores[NCH - 1].wait()


def kernel(x, wg, W1, W2):
    S0, T0, _ = x.shape
    xr = x.reshape(N, M)
    idxd3, idxc3, scale_b, cnt, laux = _gate(xr, wg)
    idxd = idxd3.reshape(NW, NCH, RCH)
    idxc = idxc3.reshape(NW, NCH, RCH)
    disp = _dispatch(xr, idxd)
    eo = _ffn(cnt, disp, W1, W2)
    combined = _combine(eo, idxc, scale_b.reshape(NW, TPW, 16)).reshape(S0, T0, M)
    return combined, laux.reshape(())
